# initial kernel scaffold (unmeasured)
import math

import jax
import jax.numpy as jnp
from jax import lax
from jax.experimental import pallas as pl
from jax.experimental.pallas import tpu as pltpu

N_DEV = 4
B, SQ, D = 2, 512, 1024
H, DH = 8, 128
ROWS = B * SQ
SCALE = 0.08838834764831843


def _body(x_ref, wq_ref, wk_ref, wv_ref, wo_ref, out_ref,
          q_s, k_s, v_s, ctx_s, comm, send_sems, recv_sems):
    my = lax.axis_index("i")
    left = lax.rem(my + N_DEV - 1, N_DEV)
    right = lax.rem(my + 1, N_DEV)

    barrier = pltpu.get_barrier_semaphore()
    for nbr in (left, right):
        pl.semaphore_signal(barrier, inc=1, device_id=(nbr,),
                            device_id_type=pl.DeviceIdType.MESH)
    pl.semaphore_wait(barrier, 2)

    row = lax.broadcasted_iota(jnp.int32, (ROWS, D), 0)
    col = lax.broadcasted_iota(jnp.int32, (ROWS, D), 1)
    s_pos = lax.rem(row, SQ).astype(jnp.float32)
    d_in_head = lax.rem(col, DH)
    pair = (d_in_head // 2).astype(jnp.float32)
    inv = jnp.exp(pair * (-2.0 * math.log(10000.0) / DH))
    ang = s_pos * inv
    cos_t = jnp.cos(ang)
    sin_t = jnp.sin(ang)
    is_even = lax.rem(col, 2) == 0

    def rope(t):
        t_next = pltpu.roll(t, -1, 1)
        t_prev = pltpu.roll(t, 1, 1)
        t_rot = jnp.where(is_even, -t_next, t_prev)
        return t * cos_t + t_rot * sin_t

    x = x_ref[...].astype(jnp.bfloat16)
    q = jnp.dot(x, wq_ref[...].astype(jnp.bfloat16),
                preferred_element_type=jnp.float32)
    q_s[...] = rope(q).astype(jnp.bfloat16)
    k = jnp.dot(x, wk_ref[...].astype(jnp.bfloat16),
                preferred_element_type=jnp.float32)
    k_s[...] = rope(k).astype(jnp.bfloat16)
    v_s[...] = jnp.dot(x, wv_ref[...].astype(jnp.bfloat16),
                       preferred_element_type=jnp.float32).astype(jnp.bfloat16)

    for b in range(B):
        for h in range(H):
            r0 = b * SQ
            c0 = h * DH
            qb = q_s[r0:r0 + SQ, c0:c0 + DH]
            kb = k_s[r0:r0 + SQ, c0:c0 + DH]
            s = lax.dot_general(qb, kb, (((1,), (1,)), ((), ())),
                                preferred_element_type=jnp.float32) * SCALE
            m = jnp.max(s, axis=1, keepdims=True)
            w = jnp.exp(s - m)
            w = w / jnp.sum(w, axis=1, keepdims=True)
            cb = jnp.dot(w.astype(jnp.bfloat16), v_s[r0:r0 + SQ, c0:c0 + DH],
                         preferred_element_type=jnp.float32)
            ctx_s[r0:r0 + SQ, c0:c0 + DH] = cb.astype(jnp.bfloat16)

    part = jnp.dot(ctx_s[...], wo_ref[...].astype(jnp.bfloat16),
                   preferred_element_type=jnp.float32)
    out_ref[...] = part
    comm[0, :, :] = part.astype(jnp.bfloat16)

    for h in range(N_DEV - 1):
        rdma = pltpu.make_async_remote_copy(
            src_ref=comm.at[h],
            dst_ref=comm.at[h + 1],
            send_sem=send_sems.at[h],
            recv_sem=recv_sems.at[h],
            device_id=(right,),
            device_id_type=pl.DeviceIdType.MESH,
        )
        rdma.start()
        rdma.wait()
        out_ref[...] += comm[h + 1, :, :].astype(jnp.float32)


def kernel(x, Wq, Wk, Wv, Wo):
    x2 = x.reshape(ROWS, D)
    out = pl.pallas_call(
        _body,
        out_shape=jax.ShapeDtypeStruct((ROWS, D), jnp.float32),
        in_specs=[pl.BlockSpec(memory_space=pltpu.VMEM)] * 5,
        out_specs=pl.BlockSpec(memory_space=pltpu.VMEM),
        scratch_shapes=[
            pltpu.VMEM((ROWS, D), jnp.bfloat16),
            pltpu.VMEM((ROWS, D), jnp.bfloat16),
            pltpu.VMEM((ROWS, D), jnp.bfloat16),
            pltpu.VMEM((ROWS, D), jnp.bfloat16),
            pltpu.VMEM((N_DEV, ROWS, D), jnp.bfloat16),
            pltpu.SemaphoreType.DMA((N_DEV - 1,)),
            pltpu.SemaphoreType.DMA((N_DEV - 1,)),
        ],
        compiler_params=pltpu.CompilerParams(collective_id=0),
    )(x2, Wq, Wk, Wv, Wo)
    return out.reshape(B, SQ, D)


# baseline (device time: 116274 ns/iter reference)
import math

import jax
import jax.numpy as jnp
from jax import lax
from jax.experimental import pallas as pl
from jax.experimental.pallas import tpu as pltpu

N_DEV = 4
B, SQ, D = 2, 512, 1024
H, DH = 8, 128
ROWS = B * SQ
SCALE = 0.08838834764831843


def _body(x_ref, wq_ref, wk_ref, wv_ref, wo_ref, out_ref,
          q_s, k_s, v_s, ctx_s, comm, send_sems, recv_sems):
    my = lax.axis_index("i")
    left = lax.rem(my + N_DEV - 1, N_DEV)
    right = lax.rem(my + 1, N_DEV)

    barrier = pltpu.get_barrier_semaphore()
    for nbr in (left, right):
        pl.semaphore_signal(barrier, inc=1, device_id=(nbr,),
                            device_id_type=pl.DeviceIdType.MESH)
    pl.semaphore_wait(barrier, 2)

    row = lax.broadcasted_iota(jnp.int32, (SQ, D), 0)
    col = lax.broadcasted_iota(jnp.int32, (SQ, D), 1)
    s_pos = row.astype(jnp.float32)
    d_in_head = lax.rem(col, DH)
    pair = (d_in_head // 2).astype(jnp.float32)
    inv = jnp.exp(pair * (-2.0 * math.log(10000.0) / DH))
    ang = s_pos * inv
    cos_t = jnp.cos(ang)
    sin_t = jnp.sin(ang)
    is_even = lax.rem(col, 2) == 0

    def rope(t):
        t_next = pltpu.roll(t, D - 1, 1)
        t_prev = pltpu.roll(t, 1, 1)
        t_rot = jnp.where(is_even, -t_next, t_prev)
        return t * cos_t + t_rot * sin_t

    wq = wq_ref[...].astype(jnp.bfloat16)
    wk = wk_ref[...].astype(jnp.bfloat16)
    wv = wv_ref[...].astype(jnp.bfloat16)
    for b in range(B):
        r0 = b * SQ
        xb = x_ref[r0:r0 + SQ, :].astype(jnp.bfloat16)
        q = jnp.dot(xb, wq, preferred_element_type=jnp.float32)
        q_s[r0:r0 + SQ, :] = rope(q).astype(jnp.bfloat16)
        k = jnp.dot(xb, wk, preferred_element_type=jnp.float32)
        k_s[r0:r0 + SQ, :] = rope(k).astype(jnp.bfloat16)
        v = jnp.dot(xb, wv, preferred_element_type=jnp.float32)
        v_s[r0:r0 + SQ, :] = v.astype(jnp.bfloat16)

    for b in range(B):
        for h in range(H):
            r0 = b * SQ
            c0 = h * DH
            qb = q_s[r0:r0 + SQ, c0:c0 + DH]
            kb = k_s[r0:r0 + SQ, c0:c0 + DH]
            s = lax.dot_general(qb, kb, (((1,), (1,)), ((), ())),
                                preferred_element_type=jnp.float32) * SCALE
            m = jnp.max(s, axis=1, keepdims=True)
            w = jnp.exp(s - m)
            w = w / jnp.sum(w, axis=1, keepdims=True)
            cb = jnp.dot(w.astype(jnp.bfloat16), v_s[r0:r0 + SQ, c0:c0 + DH],
                         preferred_element_type=jnp.float32)
            ctx_s[r0:r0 + SQ, c0:c0 + DH] = cb.astype(jnp.bfloat16)

    part = jnp.dot(ctx_s[...], wo_ref[...].astype(jnp.bfloat16),
                   preferred_element_type=jnp.float32)
    out_ref[...] = part
    comm[0, :, :] = part.astype(jnp.bfloat16)

    for h in range(N_DEV - 1):
        src = h % 2
        dst = (h + 1) % 2
        rdma = pltpu.make_async_remote_copy(
            src_ref=comm.at[src],
            dst_ref=comm.at[dst],
            send_sem=send_sems.at[h],
            recv_sem=recv_sems.at[h],
            device_id=(right,),
            device_id_type=pl.DeviceIdType.MESH,
        )
        rdma.start()
        rdma.wait()
        out_ref[...] += comm[dst, :, :].astype(jnp.float32)


def kernel(x, Wq, Wk, Wv, Wo):
    x2 = x.reshape(ROWS, D)
    out = pl.pallas_call(
        _body,
        out_shape=jax.ShapeDtypeStruct((ROWS, D), jnp.float32),
        in_specs=[pl.BlockSpec(memory_space=pltpu.VMEM)] * 5,
        out_specs=pl.BlockSpec(memory_space=pltpu.VMEM),
        scratch_shapes=[
            pltpu.VMEM((ROWS, D), jnp.bfloat16),
            pltpu.VMEM((ROWS, D), jnp.bfloat16),
            pltpu.VMEM((ROWS, D), jnp.bfloat16),
            pltpu.VMEM((ROWS, D), jnp.bfloat16),
            pltpu.VMEM((2, ROWS, D), jnp.bfloat16),
            pltpu.SemaphoreType.DMA((N_DEV - 1,)),
            pltpu.SemaphoreType.DMA((N_DEV - 1,)),
        ],
        compiler_params=pltpu.CompilerParams(collective_id=0),
    )(x2, Wq, Wk, Wv, Wo)
    return out.reshape(B, SQ, D)


# device time: 70033 ns/iter; 1.6603x vs baseline; 1.6603x over previous
import math

import jax
import jax.numpy as jnp
from jax import lax
from jax.experimental import pallas as pl
from jax.experimental.pallas import tpu as pltpu

N_DEV = 4
B, SQ, D = 2, 512, 1024
H, DH = 8, 128
ROWS = B * SQ
SCALE = 0.08838834764831843
N_CH = 4
CH = SQ // N_CH


def _body(x_ref, wq_ref, wk_ref, wv_ref, wo_ref, out_ref,
          q_s, k_s, v_s, ctx_s, bufa, bufb, rsa, rsb, aga, agb,
          send_a, recv_a, send_b, recv_b):
    my = lax.axis_index("i")
    left = lax.rem(my + N_DEV - 1, N_DEV)
    right = lax.rem(my + 1, N_DEV)

    barrier = pltpu.get_barrier_semaphore()
    for nbr in (left, right):
        pl.semaphore_signal(barrier, inc=1, device_id=(nbr,),
                            device_id_type=pl.DeviceIdType.MESH)
    pl.semaphore_wait(barrier, 2)

    row = lax.broadcasted_iota(jnp.int32, (SQ, D), 0)
    col = lax.broadcasted_iota(jnp.int32, (SQ, D), 1)
    s_pos = row.astype(jnp.float32)
    d_in_head = lax.rem(col, DH)
    pair = (d_in_head // 2).astype(jnp.float32)
    inv = jnp.exp(pair * (-2.0 * math.log(10000.0) / DH))
    ang = s_pos * inv
    cos_t = jnp.cos(ang)
    sin_t = jnp.sin(ang)
    is_even = lax.rem(col, 2) == 0

    def rope(t):
        t_next = pltpu.roll(t, D - 1, 1)
        t_prev = pltpu.roll(t, 1, 1)
        t_rot = jnp.where(is_even, -t_next, t_prev)
        return t * cos_t + t_rot * sin_t

    wq = wq_ref[...].astype(jnp.bfloat16)
    wk = wk_ref[...].astype(jnp.bfloat16)
    wv = wv_ref[...].astype(jnp.bfloat16)
    for b in range(B):
        r0 = b * SQ
        xb = x_ref[r0:r0 + SQ, :].astype(jnp.bfloat16)
        q = jnp.dot(xb, wq, preferred_element_type=jnp.float32)
        q_s[r0:r0 + SQ, :] = rope(q).astype(jnp.bfloat16)
        k = jnp.dot(xb, wk, preferred_element_type=jnp.float32)
        k_s[r0:r0 + SQ, :] = rope(k).astype(jnp.bfloat16)
        v = jnp.dot(xb, wv, preferred_element_type=jnp.float32)
        v_s[r0:r0 + SQ, :] = v.astype(jnp.bfloat16)

    for b in range(B):
        for h in range(H):
            r0 = b * SQ
            c0 = h * DH
            qb = q_s[r0:r0 + SQ, c0:c0 + DH]
            kb = k_s[r0:r0 + SQ, c0:c0 + DH]
            s = lax.dot_general(qb, kb, (((1,), (1,)), ((), ())),
                                preferred_element_type=jnp.float32) * SCALE
            m = jnp.max(s, axis=1, keepdims=True)
            w = jnp.exp(s - m)
            w = w / jnp.sum(w, axis=1, keepdims=True)
            cb = jnp.dot(w.astype(jnp.bfloat16), v_s[r0:r0 + SQ, c0:c0 + DH],
                         preferred_element_type=jnp.float32)
            ctx_s[r0:r0 + SQ, c0:c0 + DH] = cb.astype(jnp.bfloat16)

    wo = wo_ref[...].astype(jnp.bfloat16)
    for half, buf in ((0, bufa), (1, bufb)):
        r0 = half * SQ
        ph = jnp.dot(ctx_s[r0:r0 + SQ, :], wo,
                     preferred_element_type=jnp.float32).astype(jnp.bfloat16)
        for c in range(N_CH):
            buf[c, :, :] = ph[c * CH:(c + 1) * CH, :]

    def hop(ring, h):
        descs = []
        for (buf, rs_recv, ag_recv, ssem, rsem, dst_dev, sign) in (
            (bufa, rsa, aga, send_a, recv_a, right, 1),
            (bufb, rsb, agb, send_b, recv_b, left, -1),
        ):
            if ring == "rs":
                src = buf.at[lax.rem(my - sign * h + 2 * N_DEV, N_DEV)]
            else:
                src = (buf.at[lax.rem(my + sign + N_DEV, N_DEV)]
                       if h == 0 else ag_recv.at[h - 1])
            dst = (rs_recv if ring == "rs" else ag_recv).at[h]
            sem_i = h if ring == "rs" else h + 3
            descs.append(pltpu.make_async_remote_copy(
                src_ref=src, dst_ref=dst,
                send_sem=ssem.at[sem_i], recv_sem=rsem.at[sem_i],
                device_id=(dst_dev,), device_id_type=pl.DeviceIdType.MESH,
            ))
        for d in descs:
            d.start()
        for d in descs:
            d.wait()

    mod = lambda v: lax.rem(v + 4 * N_DEV, N_DEV)

    for h in range(N_DEV - 1):
        hop("rs", h)
        for (buf, rs_recv, off, sign) in ((bufa, rsa, 0, 1),
                                          (bufb, rsb, SQ, -1)):
            c_r = mod(my - sign * (1 + h))
            s = (rs_recv[h, :, :].astype(jnp.float32)
                 + buf[c_r, :, :].astype(jnp.float32))
            if h < N_DEV - 2:
                buf[c_r, :, :] = s.astype(jnp.bfloat16)
            else:
                out_ref[pl.ds(off + c_r * CH, CH), :] = s
                buf[c_r, :, :] = s.astype(jnp.bfloat16)

    for h in range(N_DEV - 1):
        hop("ag", h)
        for (ag_recv, off, sign) in ((aga, 0, 1), (agb, SQ, -1)):
            c_r = mod(my - sign * h)
            out_ref[pl.ds(off + c_r * CH, CH), :] = (
                ag_recv[h, :, :].astype(jnp.float32))


def kernel(x, Wq, Wk, Wv, Wo):
    x2 = x.reshape(ROWS, D)
    out = pl.pallas_call(
        _body,
        out_shape=jax.ShapeDtypeStruct((ROWS, D), jnp.float32),
        in_specs=[pl.BlockSpec(memory_space=pltpu.VMEM)] * 5,
        out_specs=pl.BlockSpec(memory_space=pltpu.VMEM),
        scratch_shapes=[
            pltpu.VMEM((ROWS, D), jnp.bfloat16),
            pltpu.VMEM((ROWS, D), jnp.bfloat16),
            pltpu.VMEM((ROWS, D), jnp.bfloat16),
            pltpu.VMEM((ROWS, D), jnp.bfloat16),
            pltpu.VMEM((N_CH, CH, D), jnp.bfloat16),
            pltpu.VMEM((N_CH, CH, D), jnp.bfloat16),
            pltpu.VMEM((N_DEV - 1, CH, D), jnp.bfloat16),
            pltpu.VMEM((N_DEV - 1, CH, D), jnp.bfloat16),
            pltpu.VMEM((N_DEV - 1, CH, D), jnp.bfloat16),
            pltpu.VMEM((N_DEV - 1, CH, D), jnp.bfloat16),
            pltpu.SemaphoreType.DMA((6,)),
            pltpu.SemaphoreType.DMA((6,)),
            pltpu.SemaphoreType.DMA((6,)),
            pltpu.SemaphoreType.DMA((6,)),
        ],
        compiler_params=pltpu.CompilerParams(collective_id=0),
    )(x2, Wq, Wk, Wv, Wo)
    return out.reshape(B, SQ, D)
